# baseline (speedup 1.0000x reference)
import functools

import jax
import jax.numpy as jnp
from jax import lax
from jax.experimental import pallas as pl
from jax.experimental.pallas import tpu as pltpu

_PROB_THRESHOLD = 1e-05
_NUM_OUT = 32768
_ROWS = 128
_TINY = 1.1754943508222875e-38

_KS = (1832780943, 270669613)
_KU = (64467757, 2916123636)


def _threefry_bits(k0, k1, x1):
    u32 = jnp.uint32
    k0 = u32(k0)
    k1 = u32(k1)
    k2 = u32(k0 ^ k1 ^ 0x1BD11BDA)
    x0 = k0
    rot = ((13, 15, 26, 6), (17, 29, 16, 24))
    keys = ((k1, k2), (k2, k0), (k0, k1), (k1, k2), (k2, k0))
    for i in range(5):
        for r in rot[i % 2]:
            x0 = x0 + x1
            x1 = (x1 << u32(r)) | (x1 >> u32(32 - r))
            x1 = x1 ^ x0
        ka, kb = keys[i]
        x0 = x0 + ka
        x1 = x1 + kb + u32(i + 1)
    return x0 ^ x1


def _uniform_from_bits(bits, minval):
    fb = (bits >> jnp.uint32(9)) | jnp.uint32(0x3F800000)
    f = lax.bitcast_convert_type(fb, jnp.float32) - jnp.float32(1.0)
    return f + jnp.float32(minval)


def _decode_kernel(nblocks, bcols, probs_ref, next_ref, xlast_ref,
                   best_val, best_idx):
    j = pl.program_id(0)
    p = probs_ref[...]
    xl = jnp.where(p < jnp.float32(_PROB_THRESHOLD), jnp.float32(0.0), p)
    xlast_ref[...] = xl

    row = lax.broadcasted_iota(jnp.int32, (_ROWS, bcols), 0)
    col = lax.broadcasted_iota(jnp.int32, (_ROWS, bcols), 1)
    x1 = ((row * _NUM_OUT + j * bcols + col).astype(jnp.uint32)
          + jnp.uint32(_KS[1]))
    bits = _threefry_bits(_KS[0], _KS[1], x1)
    u = _uniform_from_bits(bits, _TINY)
    score = xl / jnp.log(u)

    m = jnp.min(score, axis=1, keepdims=True)
    first = jnp.min(
        jnp.where(score == m, col, jnp.int32(_NUM_OUT)),
        axis=1, keepdims=True) + j * bcols

    @pl.when(j == 0)
    def _():
        best_val[...] = m
        best_idx[...] = first

    @pl.when(j > 0)
    def _():
        take = m < best_val[...]
        best_val[...] = jnp.where(take, m, best_val[...])
        best_idx[...] = jnp.where(take, first, best_idx[...])

    @pl.when(j == nblocks - 1)
    def _():
        idx = best_idx[...]
        ucnt = (lax.broadcasted_iota(jnp.int32, (_ROWS, 1), 0).astype(jnp.uint32)
                + jnp.uint32(_KU[1]))
        ubits = _threefry_bits(_KU[0], _KU[1], ucnt)
        noise = _uniform_from_bits(ubits, 0.0)
        nt = (idx.astype(jnp.float32) + noise) * jnp.float32(1.0 / _NUM_OUT)
        next_ref[...] = jnp.where(idx == 0, jnp.float32(0.0), nt)


@jax.jit
def kernel(probs):
    nblocks = 16
    bcols = _NUM_OUT // nblocks
    next_token, x_last = pl.pallas_call(
        functools.partial(_decode_kernel, nblocks, bcols),
        grid=(nblocks,),
        in_specs=[pl.BlockSpec((_ROWS, bcols), lambda j: (0, j))],
        out_specs=[
            pl.BlockSpec((_ROWS, 1), lambda j: (0, 0)),
            pl.BlockSpec((_ROWS, bcols), lambda j: (0, j)),
        ],
        out_shape=[
            jax.ShapeDtypeStruct((_ROWS, 1), jnp.float32),
            jax.ShapeDtypeStruct((_ROWS, _NUM_OUT), jnp.float32),
        ],
        scratch_shapes=[
            pltpu.VMEM((_ROWS, 1), jnp.float32),
            pltpu.VMEM((_ROWS, 1), jnp.int32),
        ],
    )(probs)
    return next_token, x_last


# speedup vs baseline: 1.0135x; 1.0135x over previous
import functools

import jax
import jax.numpy as jnp
from jax import lax
from jax.experimental import pallas as pl
from jax.experimental.pallas import tpu as pltpu

_PROB_THRESHOLD = 1e-05
_NUM_OUT = 32768
_ROWS = 128
_TINY = 1.1754943508222875e-38

_KS = (1832780943, 270669613)
_KU = (64467757, 2916123636)


def _threefry_bits(k0, k1, x1):
    u32 = jnp.uint32
    k0 = u32(k0)
    k1 = u32(k1)
    k2 = u32(k0 ^ k1 ^ 0x1BD11BDA)
    x0 = k0
    rot = ((13, 15, 26, 6), (17, 29, 16, 24))
    keys = ((k1, k2), (k2, k0), (k0, k1), (k1, k2), (k2, k0))
    for i in range(5):
        for r in rot[i % 2]:
            x0 = x0 + x1
            x1 = (x1 << u32(r)) | (x1 >> u32(32 - r))
            x1 = x1 ^ x0
        ka, kb = keys[i]
        x0 = x0 + ka
        x1 = x1 + kb + u32(i + 1)
    return x0 ^ x1


def _uniform_from_bits(bits, minval):
    fb = (bits >> jnp.uint32(9)) | jnp.uint32(0x3F800000)
    f = lax.bitcast_convert_type(fb, jnp.float32) - jnp.float32(1.0)
    return f + jnp.float32(minval)


def _decode_kernel(nblocks, bcols, probs_ref, next_ref, xlast_ref,
                   best_val, best_idx):
    j = pl.program_id(0)
    p = probs_ref[...]
    xl = jnp.where(p < jnp.float32(_PROB_THRESHOLD), jnp.float32(0.0), p)
    xlast_ref[...] = xl

    row = lax.broadcasted_iota(jnp.int32, (_ROWS, bcols), 0)
    col = lax.broadcasted_iota(jnp.int32, (_ROWS, bcols), 1)
    x1 = ((row * _NUM_OUT + j * bcols + col).astype(jnp.uint32)
          + jnp.uint32(_KS[1]))
    bits = _threefry_bits(_KS[0], _KS[1], x1)
    u = _uniform_from_bits(bits, _TINY)
    score = xl / (-jnp.log(u))

    m = jnp.max(score, axis=1, keepdims=True)
    first = jnp.min(
        jnp.where(score == m, col, jnp.int32(_NUM_OUT)),
        axis=1, keepdims=True) + j * bcols

    @pl.when(j == 0)
    def _():
        best_val[...] = m
        best_idx[...] = first

    @pl.when(j > 0)
    def _():
        take = m > best_val[...]
        best_val[...] = jnp.where(take, m, best_val[...])
        best_idx[...] = jnp.where(take, first, best_idx[...])

    @pl.when(j == nblocks - 1)
    def _():
        idx = best_idx[...]
        ucnt = (lax.broadcasted_iota(jnp.int32, (_ROWS, 1), 0).astype(jnp.uint32)
                + jnp.uint32(_KU[1]))
        ubits = _threefry_bits(_KU[0], _KU[1], ucnt)
        noise = _uniform_from_bits(ubits, 0.0)
        nt = (idx.astype(jnp.float32) + noise) * jnp.float32(1.0 / _NUM_OUT)
        next_ref[...] = jnp.where(idx == 0, jnp.float32(0.0), nt)


@jax.jit
def kernel(probs):
    nblocks = 16
    bcols = _NUM_OUT // nblocks
    next_token, x_last = pl.pallas_call(
        functools.partial(_decode_kernel, nblocks, bcols),
        grid=(nblocks,),
        in_specs=[pl.BlockSpec((_ROWS, bcols), lambda j: (0, j))],
        out_specs=[
            pl.BlockSpec((_ROWS, 1), lambda j: (0, 0)),
            pl.BlockSpec((_ROWS, bcols), lambda j: (0, j)),
        ],
        out_shape=[
            jax.ShapeDtypeStruct((_ROWS, 1), jnp.float32),
            jax.ShapeDtypeStruct((_ROWS, _NUM_OUT), jnp.float32),
        ],
        scratch_shapes=[
            pltpu.VMEM((_ROWS, 1), jnp.float32),
            pltpu.VMEM((_ROWS, 1), jnp.int32),
        ],
    )(probs)
    return next_token, x_last
